# baseline (speedup 1.0000x reference)
import jax
import jax.numpy as jnp
from jax import lax
from jax.experimental import pallas as pl
from jax.experimental.pallas import tpu as pltpu
from jax.experimental.pallas import tpu_sc as plsc

DIM = 64
NWORKERS = 32
W = 256


def kernel(x, table):
    B, S = x.shape
    n = B * S
    v = table.shape[0]
    idx = x.reshape(n)
    b_per_w = n // NWORKERS
    steps = b_per_w // W

    mesh = plsc.VectorSubcoreMesh(core_axis_name="c", subcore_axis_name="s")
    cp = pltpu.CompilerParams(use_tc_tiling_on_sc=False)

    @pl.kernel(
        out_type=jax.ShapeDtypeStruct((n, DIM), table.dtype),
        mesh=mesh,
        scratch_types=[
            pltpu.VMEM((W,), jnp.int32),
            pltpu.VMEM((W, DIM), jnp.float32),
            pltpu.SemaphoreType.DMA,
        ],
        compiler_params=cp,
    )
    def gather_kernel(table_hbm, i_hbm, o_hbm, idx_v, gbuf, sem):
        wid = lax.axis_index("s") * 2 + lax.axis_index("c")

        @pl.loop(0, steps)
        def _(c):
            base = wid * b_per_w + c * W
            pltpu.sync_copy(i_hbm.at[pl.ds(base, W)], idx_v)
            pltpu.async_copy(table_hbm.at[idx_v], gbuf, sem).wait()
            pltpu.sync_copy(gbuf, o_hbm.at[pl.ds(base, W)])

    out = gather_kernel(table, idx)
    return out.reshape(B, S, DIM)
